# Initial kernel scaffold; baseline (speedup 1.0000x reference)
import jax
import jax.numpy as jnp
from jax.experimental import pallas as pl


def kernel(tables, sigma, base_obs, actions, responses, t):
    raise NotImplementedError("write your pallas kernel here")


# speedup vs baseline: 9.7295x; 9.7295x over previous
import functools

import jax
import jax.numpy as jnp
from jax.experimental import pallas as pl
from jax.experimental.pallas import tpu as pltpu

Y = 64
SUB, LANE = 256, 128
NCON = 9


def _row_map(b, req_row, req_val, *, j):
    return (b, req_row[b, j], 0, 0)


def _sig_map(b, req_row, req_val):
    return (b, 0, 0)


def _out_map(b, req_row, req_val):
    return (b, 0, 0)


def _hist_kernel(req_row, req_val, *refs):
    rows = refs[:NCON]
    sig_ref = refs[NCON]
    out_ref = refs[NCON + 1]
    b = pl.program_id(0)
    m = rows[0][0, 0] == req_val[b, 0]
    for j in range(1, NCON):
        m = jnp.logical_and(m, rows[j][0, 0] == req_val[b, j])
    ms = jnp.where(m, sig_ref[0], Y)
    cnt = jnp.stack(
        [jnp.sum((ms == c).astype(jnp.float32)) for c in range(Y)]
    ).reshape(1, Y)
    z = jnp.maximum(jnp.sum(cnt), 1.0)
    out_ref[0] = jnp.log(jnp.maximum(cnt / z, 1e-9))


def kernel(tables, sigma, base_obs, actions, responses, t):
    B, V, N = tables.shape
    T = actions.shape[1]
    assert N == SUB * LANE

    actions = actions.astype(jnp.int32)
    responses = responses.astype(jnp.int32)
    base_obs = base_obs.astype(jnp.int32)

    active = jnp.arange(T, dtype=jnp.int32)[None, :] < t
    use_real = active & (actions != V)
    a_c = jnp.clip(actions, 0, V - 1)
    req_row = jnp.concatenate(
        [jnp.zeros((B, 1), jnp.int32), jnp.where(use_real, a_c, 0)], axis=1
    )
    req_val = jnp.concatenate(
        [base_obs[:, None], jnp.where(use_real, responses, base_obs[:, None])],
        axis=1,
    )

    t4 = tables.reshape(B, V, SUB, LANE)
    s3 = sigma.reshape(B, SUB, LANE)

    grid_spec = pltpu.PrefetchScalarGridSpec(
        num_scalar_prefetch=2,
        grid=(B,),
        in_specs=(
            [
                pl.BlockSpec((1, 1, SUB, LANE), functools.partial(_row_map, j=j))
                for j in range(NCON)
            ]
            + [pl.BlockSpec((1, SUB, LANE), _sig_map)]
        ),
        out_specs=pl.BlockSpec((1, 1, Y), _out_map),
    )
    out = pl.pallas_call(
        _hist_kernel,
        grid_spec=grid_spec,
        out_shape=jax.ShapeDtypeStruct((B, 1, Y), jnp.float32),
    )(req_row, req_val, *([t4] * NCON), s3)
    return out.reshape(B, Y)


# speedup vs baseline: 9.7307x; 1.0001x over previous
import functools

import jax
import jax.numpy as jnp
from jax.experimental import pallas as pl
from jax.experimental.pallas import tpu as pltpu

Y = 64
SUB, LANE = 256, 128
NCON = 9


def _row_map(b, req_row, req_val, *, j):
    return (b, req_row[b, j], 0, 0)


def _sig_map(b, req_row, req_val):
    return (b, 0, 0)


def _out_map(b, req_row, req_val):
    return (b, 0, 0)


def _hist_kernel(req_row, req_val, *refs):
    rows = refs[:NCON]
    sig_ref = refs[NCON]
    out_ref = refs[NCON + 1]
    b = pl.program_id(0)
    m = rows[0][0, 0] == req_val[b, 0]
    for j in range(1, NCON):
        m = jnp.logical_and(m, rows[j][0, 0] == req_val[b, j])
    ms = jnp.where(m, sig_ref[0], Y)
    cnt = jnp.stack(
        [jnp.sum((ms == c).astype(jnp.float32)) for c in range(Y)]
    ).reshape(1, Y)
    z = jnp.maximum(jnp.sum(cnt), 1.0)
    out_ref[0] = jnp.log(jnp.maximum(cnt / z, 1e-9))


def kernel(tables, sigma, base_obs, actions, responses, t):
    B, V, N = tables.shape
    T = actions.shape[1]
    assert N == SUB * LANE

    actions = actions.astype(jnp.int32)
    responses = responses.astype(jnp.int32)
    base_obs = base_obs.astype(jnp.int32)

    active = jnp.arange(T, dtype=jnp.int32)[None, :] < t
    use_real = active & (actions != V)
    a_c = jnp.clip(actions, 0, V - 1)
    req_row = jnp.concatenate(
        [jnp.zeros((B, 1), jnp.int32), jnp.where(use_real, a_c, 0)], axis=1
    )
    req_val = jnp.concatenate(
        [base_obs[:, None], jnp.where(use_real, responses, base_obs[:, None])],
        axis=1,
    )

    t4 = tables.reshape(B, V, SUB, LANE)
    s3 = sigma.reshape(B, SUB, LANE)

    grid_spec = pltpu.PrefetchScalarGridSpec(
        num_scalar_prefetch=2,
        grid=(B,),
        in_specs=(
            [
                pl.BlockSpec((1, 1, SUB, LANE), functools.partial(_row_map, j=j))
                for j in range(NCON)
            ]
            + [pl.BlockSpec((1, SUB, LANE), _sig_map)]
        ),
        out_specs=pl.BlockSpec((1, 1, Y), _out_map),
    )
    out = pl.pallas_call(
        _hist_kernel,
        grid_spec=grid_spec,
        out_shape=jax.ShapeDtypeStruct((B, 1, Y), jnp.float32),
        compiler_params=pltpu.CompilerParams(
            dimension_semantics=("parallel",)
        ),
    )(req_row, req_val, *([t4] * NCON), s3)
    return out.reshape(B, Y)


# speedup vs baseline: 15.5730x; 1.6004x over previous
import jax
import jax.numpy as jnp
from jax.experimental import pallas as pl
from jax.experimental.pallas import tpu as pltpu

Y = 64
SUB, LANE = 256, 128


def _tab_map(b, ncon):
    return (b, 0, 0)


def _req_map(b, ncon):
    return (b, 0, 0)


def _sig_map(b, ncon):
    return (b, 0, 0)


def _out_map(b, ncon):
    return (b, 0, 0)


def _hist_kernel(ncon_ref, tab_ref, req_ref, sig_ref, out_ref):
    b = pl.program_id(0)
    tab = tab_ref[0]
    req = req_ref[0]
    eq = (tab == req).astype(jnp.int32)
    cnt = jnp.sum(eq, axis=0, keepdims=True)
    key = cnt.reshape(SUB, LANE) * Y + sig_ref[0] - ncon_ref[b] * Y
    hist = jnp.stack(
        [jnp.sum((key == c).astype(jnp.float32)) for c in range(Y)]
    ).reshape(1, Y)
    z = jnp.maximum(jnp.sum(hist), 1.0)
    out_ref[0] = jnp.log(jnp.maximum(hist / z, 1e-9))


def kernel(tables, sigma, base_obs, actions, responses, t):
    B, V, N = tables.shape
    T = actions.shape[1]
    assert N == SUB * LANE

    actions = actions.astype(jnp.int32)
    responses = responses.astype(jnp.int32)
    base_obs = base_obs.astype(jnp.int32)

    active = jnp.arange(T, dtype=jnp.int32)[None, :] < t
    use_real = active & (actions != V)
    a_c = jnp.clip(actions, 0, V - 1)
    rows = jnp.concatenate(
        [jnp.zeros((B, 1), jnp.int32), jnp.where(use_real, a_c, 0)], axis=1
    )
    vals = jnp.concatenate(
        [base_obs[:, None], jnp.where(use_real, responses, base_obs[:, None])],
        axis=1,
    )

    BIG = jnp.int32(1 << 20)
    hit = rows[:, None, :] == jnp.arange(V, dtype=jnp.int32)[None, :, None]
    vmin = jnp.min(jnp.where(hit, vals[:, None, :], BIG), axis=2)
    vmax = jnp.max(jnp.where(hit, vals[:, None, :], -BIG), axis=2)
    con = jnp.any(hit, axis=2)
    req = jnp.where(con, vmin, -1).astype(jnp.int32)
    feasible = jnp.all(~con | (vmin == vmax), axis=1)
    ncon = jnp.where(
        feasible, jnp.sum(con.astype(jnp.int32), axis=1), V + 1
    ).astype(jnp.int32)

    s3 = sigma.reshape(B, SUB, LANE)
    req3 = req[:, :, None]

    grid_spec = pltpu.PrefetchScalarGridSpec(
        num_scalar_prefetch=1,
        grid=(B,),
        in_specs=(
            pl.BlockSpec((1, V, N), _tab_map),
            pl.BlockSpec((1, V, 1), _req_map),
            pl.BlockSpec((1, SUB, LANE), _sig_map),
        ),
        out_specs=pl.BlockSpec((1, 1, Y), _out_map),
    )
    out = pl.pallas_call(
        _hist_kernel,
        grid_spec=grid_spec,
        out_shape=jax.ShapeDtypeStruct((B, 1, Y), jnp.float32),
        compiler_params=pltpu.CompilerParams(
            dimension_semantics=("arbitrary",)
        ),
    )(ncon, tables, req3, s3)
    return out.reshape(B, Y)


# speedup vs baseline: 16.4837x; 1.0585x over previous
import jax
import jax.numpy as jnp
from jax.experimental import pallas as pl
from jax.experimental.pallas import tpu as pltpu

Y = 64
SUB, LANE = 256, 128


def _tab_map(b, ncon):
    return (b, 0, 0)


def _req_map(b, ncon):
    return (b, 0, 0)


def _sig_map(b, ncon):
    return (b, 0, 0)


def _out_map(b, ncon):
    return (b, 0, 0)


def _hist_kernel(ncon_ref, tab_ref, req_ref, sig_ref, out_ref):
    b = pl.program_id(0)
    tab = tab_ref[0]
    req = req_ref[0]
    eq = (tab == req).astype(jnp.float32)
    cnt = jax.lax.dot_general(
        jnp.ones((1, tab.shape[0]), jnp.float32),
        eq,
        (((1,), (0,)), ((), ())),
        preferred_element_type=jnp.float32,
    )
    key = (
        cnt.reshape(SUB, LANE) * Y
        + sig_ref[0].astype(jnp.float32)
        - (ncon_ref[b] * Y).astype(jnp.float32)
    )
    hist = jnp.stack(
        [jnp.sum((key == c).astype(jnp.float32)) for c in range(Y)]
    ).reshape(1, Y)
    z = jnp.maximum(jnp.sum(hist), 1.0)
    out_ref[0] = jnp.log(jnp.maximum(hist / z, 1e-9))


def kernel(tables, sigma, base_obs, actions, responses, t):
    B, V, N = tables.shape
    T = actions.shape[1]
    assert N == SUB * LANE

    actions = actions.astype(jnp.int32)
    responses = responses.astype(jnp.int32)
    base_obs = base_obs.astype(jnp.int32)

    active = jnp.arange(T, dtype=jnp.int32)[None, :] < t
    use_real = active & (actions != V)
    a_c = jnp.clip(actions, 0, V - 1)
    rows = jnp.concatenate(
        [jnp.zeros((B, 1), jnp.int32), jnp.where(use_real, a_c, 0)], axis=1
    )
    vals = jnp.concatenate(
        [base_obs[:, None], jnp.where(use_real, responses, base_obs[:, None])],
        axis=1,
    )

    BIG = jnp.int32(1 << 20)
    hit = rows[:, None, :] == jnp.arange(V, dtype=jnp.int32)[None, :, None]
    vmin = jnp.min(jnp.where(hit, vals[:, None, :], BIG), axis=2)
    vmax = jnp.max(jnp.where(hit, vals[:, None, :], -BIG), axis=2)
    con = jnp.any(hit, axis=2)
    req = jnp.where(con, vmin, -1).astype(jnp.int32)
    feasible = jnp.all(~con | (vmin == vmax), axis=1)
    ncon = jnp.where(
        feasible, jnp.sum(con.astype(jnp.int32), axis=1), V + 1
    ).astype(jnp.int32)

    s3 = sigma.reshape(B, SUB, LANE)
    req3 = req[:, :, None]

    grid_spec = pltpu.PrefetchScalarGridSpec(
        num_scalar_prefetch=1,
        grid=(B,),
        in_specs=(
            pl.BlockSpec((1, V, N), _tab_map),
            pl.BlockSpec((1, V, 1), _req_map),
            pl.BlockSpec((1, SUB, LANE), _sig_map),
        ),
        out_specs=pl.BlockSpec((1, 1, Y), _out_map),
    )
    out = pl.pallas_call(
        _hist_kernel,
        grid_spec=grid_spec,
        out_shape=jax.ShapeDtypeStruct((B, 1, Y), jnp.float32),
        compiler_params=pltpu.CompilerParams(
            dimension_semantics=("arbitrary",)
        ),
    )(ncon, tables, req3, s3)
    return out.reshape(B, Y)
